# baseline (speedup 1.0000x reference)
import jax
import jax.numpy as jnp
from jax import lax
from jax.experimental import pallas as pl
from jax.experimental.pallas import tpu as pltpu
from jax.experimental.pallas import tpu_sc as plsc

B = 16384
D = 512
C = 100000

_INFO = plsc.get_sparse_core_info()
NC = _INFO.num_cores
NS = _INFO.num_subcores
L = _INFO.num_lanes
NW = NC * NS

PW = B // NW
PH = B // NS
HCHUNK = 128
NHC = PH // HCHUNK
HIST = 100352
ZS = HIST // NS
CH = 16
NCHK = PW // CH
NBUF = 7
LOOK = 5
DB = D // L


def _body(feat_hbm, y_hbm, centers_hbm, out_hbm,
          hist_sh, yh_v, idx_v, cnt_v, zbuf, ones_v, rows3, feat3,
          acc_buf, sem_h, sem_r, sem_f):
    c = lax.axis_index("c")
    s = lax.axis_index("s")
    wid = s * NC + c
    base = wid * PW

    def start(g, b):
        pltpu.async_copy(centers_hbm.at[idx_v.at[pl.ds(g * CH, CH)]],
                         rows3.at[b], sem_r.at[b])
        pltpu.async_copy(feat_hbm.at[pl.ds(base + g * CH, CH)],
                         feat3.at[b], sem_f.at[b])

    def wait(g, b):
        pltpu.make_async_copy(centers_hbm.at[idx_v.at[pl.ds(g * CH, CH)]],
                              rows3.at[b], sem_r.at[b]).wait()
        pltpu.make_async_copy(feat_hbm.at[pl.ds(base + g * CH, CH)],
                              feat3.at[b], sem_f.at[b]).wait()

    pltpu.sync_copy(y_hbm.at[pl.ds(wid * PW, PW)], idx_v)
    for g0 in range(LOOK):
        start(g0, g0)

    for j in range(NHC):
        pltpu.async_copy(y_hbm.at[pl.ds(s * PH + j * HCHUNK, HCHUNK)],
                         yh_v.at[j], sem_h)
    for j in range(NHC):
        pltpu.make_async_copy(y_hbm.at[pl.ds(s * PH + j * HCHUNK, HCHUNK)],
                              yh_v.at[j], sem_h).wait()

    zero16 = jnp.zeros((L,), jnp.float32)
    one16 = jnp.ones((L,), jnp.float32)

    def zfill(i, _):
        for u in range(8):
            zbuf[pl.ds(i * 8 * L + u * L, L)] = zero16
        return 0
    lax.fori_loop(0, ZS // (8 * L), zfill, 0)

    def ofill(i, _):
        ones_v[pl.ds(i * L, L)] = one16
        return 0
    lax.fori_loop(0, HCHUNK // L, ofill, 0)

    pltpu.sync_copy(zbuf, hist_sh.at[pl.ds(s * ZS, ZS)])
    plsc.subcore_barrier()

    for j in range(NHC):
        pltpu.async_copy(ones_v, hist_sh.at[yh_v.at[j]], sem_h, add=True)
    for j in range(NHC):
        pltpu.make_async_copy(ones_v, hist_sh.at[yh_v.at[j]], sem_h).wait()
    plsc.subcore_barrier()

    for j in range(PW // HCHUNK):
        pltpu.async_copy(hist_sh.at[idx_v.at[pl.ds(j * HCHUNK, HCHUNK)]],
                         cnt_v.at[pl.ds(j * HCHUNK, HCHUNK)], sem_h)
    for j in range(PW // HCHUNK):
        pltpu.make_async_copy(
            hist_sh.at[idx_v.at[pl.ds(j * HCHUNK, HCHUNK)]],
            cnt_v.at[pl.ds(j * HCHUNK, HCHUNK)], sem_h).wait()

    def invf(i, _):
        v = cnt_v[pl.ds(i * L, L)]
        cnt_v[pl.ds(i * L, L)] = 1.0 / (v + 1.0)
        return 0
    lax.fori_loop(0, PW // L, invf, 0)

    def chunk_compute(g, b, a):
        def row_body(r, a2):
            inv_b = plsc.load_gather(
                cnt_v, [jnp.full((L,), g * CH + r, jnp.int32)])
            accs = [jnp.zeros((L,), jnp.float32) for _ in range(4)]
            for kd in range(DB):
                f = feat3[b, r, pl.ds(kd * L, L)]
                cc = rows3[b, r, pl.ds(kd * L, L)]
                d = f - cc
                accs[kd % 4] = accs[kd % 4] + d * d
            s4 = (accs[0] + accs[1]) + (accs[2] + accs[3])
            return a2 + s4 * inv_b

        return lax.fori_loop(0, CH, row_body, a)

    def round_body(g, a):
        b = lax.rem(g, NBUF)
        wait(g, b)
        nxt = g + LOOK

        @pl.when(nxt < NCHK)
        def _():
            start(nxt, lax.rem(nxt, NBUF))

        return chunk_compute(g, b, a)

    acc = lax.fori_loop(0, NCHK, round_body, jnp.zeros((L,), jnp.float32))

    acc_buf[...] = acc
    pltpu.sync_copy(acc_buf, out_hbm.at[wid])


@jax.jit
def _sc_center_loss(feat, y, centers):
    mesh = plsc.VectorSubcoreMesh(core_axis_name="c", subcore_axis_name="s")
    run = pl.kernel(
        _body,
        out_type=jax.ShapeDtypeStruct((NW, L), jnp.float32),
        mesh=mesh,
        scratch_types=[
            pltpu.VMEM_SHARED((HIST,), jnp.float32),
            pltpu.VMEM((NHC, HCHUNK), jnp.int32),
            pltpu.VMEM((PW,), jnp.int32),
            pltpu.VMEM((PW,), jnp.float32),
            pltpu.VMEM((ZS,), jnp.float32),
            pltpu.VMEM((HCHUNK,), jnp.float32),
            pltpu.VMEM((NBUF, CH, D), jnp.float32),
            pltpu.VMEM((NBUF, CH, D), jnp.float32),
            pltpu.VMEM((L,), jnp.float32),
            pltpu.SemaphoreType.DMA,
            pltpu.SemaphoreType.DMA((NBUF,)),
            pltpu.SemaphoreType.DMA((NBUF,)),
        ],
        compiler_params=pltpu.CompilerParams(needs_layout_passes=False),
    )
    return run(feat, y, centers)


def kernel(feat, y, centers):
    partials = _sc_center_loss(feat, y, centers)
    return 0.5 * jnp.sum(partials)


# speedup vs baseline: 1.0110x; 1.0110x over previous
import jax
import jax.numpy as jnp
from jax import lax
from jax.experimental import pallas as pl
from jax.experimental.pallas import tpu as pltpu
from jax.experimental.pallas import tpu_sc as plsc

B = 16384
D = 512
C = 100000

_INFO = plsc.get_sparse_core_info()
NC = _INFO.num_cores
NS = _INFO.num_subcores
L = _INFO.num_lanes
NW = NC * NS

PW = B // NW
PH = B // NS
HCHUNK = 128
NHC = PH // HCHUNK
HIST = 100352
ZS = HIST // NS
CH = 16
NCHK = PW // CH
NBUF = 6
LOOK = 4
DB = D // L


def _body(feat_hbm, y_hbm, centers_hbm, out_hbm,
          hist_sh, yh_v, idx_v, cnt_v, zbuf, ones_v, rows3, feat3,
          acc_buf, sem_h, sem_r, sem_f):
    c = lax.axis_index("c")
    s = lax.axis_index("s")
    wid = s * NC + c
    base = wid * PW

    def start(g, b):
        pltpu.async_copy(centers_hbm.at[idx_v.at[pl.ds(g * CH, CH)]],
                         rows3.at[b], sem_r.at[b])
        pltpu.async_copy(feat_hbm.at[pl.ds(base + g * CH, CH)],
                         feat3.at[b], sem_f.at[b])

    def wait(g, b):
        pltpu.make_async_copy(centers_hbm.at[idx_v.at[pl.ds(g * CH, CH)]],
                              rows3.at[b], sem_r.at[b]).wait()
        pltpu.make_async_copy(feat_hbm.at[pl.ds(base + g * CH, CH)],
                              feat3.at[b], sem_f.at[b]).wait()

    pltpu.sync_copy(y_hbm.at[pl.ds(wid * PW, PW)], idx_v)
    for g0 in range(LOOK):
        start(g0, g0)

    for j in range(NHC):
        pltpu.async_copy(y_hbm.at[pl.ds(s * PH + j * HCHUNK, HCHUNK)],
                         yh_v.at[j], sem_h)
    for j in range(NHC):
        pltpu.make_async_copy(y_hbm.at[pl.ds(s * PH + j * HCHUNK, HCHUNK)],
                              yh_v.at[j], sem_h).wait()

    zero16 = jnp.zeros((L,), jnp.float32)
    one16 = jnp.ones((L,), jnp.float32)

    def zfill(i, _):
        for u in range(8):
            zbuf[pl.ds(i * 8 * L + u * L, L)] = zero16
        return 0
    lax.fori_loop(0, ZS // (8 * L), zfill, 0)

    def ofill(i, _):
        ones_v[pl.ds(i * L, L)] = one16
        return 0
    lax.fori_loop(0, HCHUNK // L, ofill, 0)

    pltpu.sync_copy(zbuf, hist_sh.at[pl.ds(s * ZS, ZS)])
    plsc.subcore_barrier()

    for j in range(NHC):
        pltpu.async_copy(ones_v, hist_sh.at[yh_v.at[j]], sem_h, add=True)
    for j in range(NHC):
        pltpu.make_async_copy(ones_v, hist_sh.at[yh_v.at[j]], sem_h).wait()
    plsc.subcore_barrier()

    for j in range(PW // HCHUNK):
        pltpu.async_copy(hist_sh.at[idx_v.at[pl.ds(j * HCHUNK, HCHUNK)]],
                         cnt_v.at[pl.ds(j * HCHUNK, HCHUNK)], sem_h)
    for j in range(PW // HCHUNK):
        pltpu.make_async_copy(
            hist_sh.at[idx_v.at[pl.ds(j * HCHUNK, HCHUNK)]],
            cnt_v.at[pl.ds(j * HCHUNK, HCHUNK)], sem_h).wait()

    def invf(i, _):
        v = cnt_v[pl.ds(i * L, L)]
        cnt_v[pl.ds(i * L, L)] = 1.0 / (v + 1.0)
        return 0
    lax.fori_loop(0, PW // L, invf, 0)

    def chunk_compute(g, b, a):
        def row_body(r, a2):
            inv_b = plsc.load_gather(
                cnt_v, [jnp.full((L,), g * CH + r, jnp.int32)])
            accs = [jnp.zeros((L,), jnp.float32) for _ in range(4)]
            for kd in range(DB):
                f = feat3[b, r, pl.ds(kd * L, L)]
                cc = rows3[b, r, pl.ds(kd * L, L)]
                d = f - cc
                accs[kd % 4] = accs[kd % 4] + d * d
            s4 = (accs[0] + accs[1]) + (accs[2] + accs[3])
            return a2 + s4 * inv_b

        return lax.fori_loop(0, CH, row_body, a)

    def round_body(g, a):
        b = lax.rem(g, NBUF)
        wait(g, b)
        nxt = g + LOOK

        @pl.when(nxt < NCHK)
        def _():
            start(nxt, lax.rem(nxt, NBUF))

        return chunk_compute(g, b, a)

    acc = lax.fori_loop(0, NCHK, round_body, jnp.zeros((L,), jnp.float32))

    acc_buf[...] = acc
    pltpu.sync_copy(acc_buf, out_hbm.at[wid])


@jax.jit
def _sc_center_loss(feat, y, centers):
    mesh = plsc.VectorSubcoreMesh(core_axis_name="c", subcore_axis_name="s")
    run = pl.kernel(
        _body,
        out_type=jax.ShapeDtypeStruct((NW, L), jnp.float32),
        mesh=mesh,
        scratch_types=[
            pltpu.VMEM_SHARED((HIST,), jnp.float32),
            pltpu.VMEM((NHC, HCHUNK), jnp.int32),
            pltpu.VMEM((PW,), jnp.int32),
            pltpu.VMEM((PW,), jnp.float32),
            pltpu.VMEM((ZS,), jnp.float32),
            pltpu.VMEM((HCHUNK,), jnp.float32),
            pltpu.VMEM((NBUF, CH, D), jnp.float32),
            pltpu.VMEM((NBUF, CH, D), jnp.float32),
            pltpu.VMEM((L,), jnp.float32),
            pltpu.SemaphoreType.DMA,
            pltpu.SemaphoreType.DMA((NBUF,)),
            pltpu.SemaphoreType.DMA((NBUF,)),
        ],
        compiler_params=pltpu.CompilerParams(needs_layout_passes=False),
    )
    return run(feat, y, centers)


def kernel(feat, y, centers):
    partials = _sc_center_loss(feat, y, centers)
    return 0.5 * jnp.sum(partials)


# speedup vs baseline: 1.0693x; 1.0577x over previous
import jax
import jax.numpy as jnp
from jax import lax
from jax.experimental import pallas as pl
from jax.experimental.pallas import tpu as pltpu
from jax.experimental.pallas import tpu_sc as plsc

B = 16384
D = 512
C = 100000

_INFO = plsc.get_sparse_core_info()
NC = _INFO.num_cores
NS = _INFO.num_subcores
L = _INFO.num_lanes
NW = NC * NS

PW = B // NW
PH = B // NS
HCHUNK = 128
NHC = PH // HCHUNK
HIST = 100352
ZS = HIST // NS
CH = 16
NCHK = PW // CH
NBUF = 6
LOOK = 4
RPC = HCHUNK // CH
DB = D // L


def _body(feat_hbm, y2_hbm, centers_hbm, out_hbm,
          hist_sh, yh_v, cnt_v, zbuf, ones_v, rows3, feat3,
          acc_buf, sem_h, sem_r, sem_f):
    c = lax.axis_index("c")
    s = lax.axis_index("s")
    wid = s * NC + c
    base = wid * PW

    def idx_ref(g):
        return yh_v.at[c * (PW // HCHUNK) + g // RPC,
                       pl.ds(lax.rem(g, RPC) * CH, CH)]

    def start(g, b):
        pltpu.async_copy(centers_hbm.at[idx_ref(g)], rows3.at[b],
                         sem_r.at[b])
        pltpu.async_copy(feat_hbm.at[pl.ds(base + g * CH, CH)],
                         feat3.at[b], sem_f.at[b])

    def wait(g, b):
        pltpu.make_async_copy(centers_hbm.at[idx_ref(g)], rows3.at[b],
                              sem_r.at[b]).wait()
        pltpu.make_async_copy(feat_hbm.at[pl.ds(base + g * CH, CH)],
                              feat3.at[b], sem_f.at[b]).wait()

    pltpu.sync_copy(y2_hbm.at[pl.ds(s * NHC, NHC)], yh_v)
    for g0 in range(LOOK):
        start(g0, g0)

    zero16 = jnp.zeros((L,), jnp.float32)
    one16 = jnp.ones((L,), jnp.float32)

    def zfill(i, _):
        for u in range(8):
            zbuf[pl.ds(i * 8 * L + u * L, L)] = zero16
        return 0
    lax.fori_loop(0, ZS // (8 * L), zfill, 0)

    def ofill(i, _):
        ones_v[pl.ds(i * L, L)] = one16
        return 0
    lax.fori_loop(0, HCHUNK // L, ofill, 0)

    pltpu.sync_copy(zbuf, hist_sh.at[pl.ds(s * ZS, ZS)])
    plsc.subcore_barrier()

    for j in range(NHC):
        pltpu.async_copy(ones_v, hist_sh.at[yh_v.at[j]], sem_h, add=True)
    for j in range(NHC):
        pltpu.make_async_copy(ones_v, hist_sh.at[yh_v.at[j]], sem_h).wait()
    plsc.subcore_barrier()

    for j in range(PW // HCHUNK):
        pltpu.async_copy(hist_sh.at[yh_v.at[c * (PW // HCHUNK) + j]],
                         cnt_v.at[pl.ds(j * HCHUNK, HCHUNK)], sem_h)
    for j in range(PW // HCHUNK):
        pltpu.make_async_copy(
            hist_sh.at[yh_v.at[c * (PW // HCHUNK) + j]],
            cnt_v.at[pl.ds(j * HCHUNK, HCHUNK)], sem_h).wait()

    def invf(i, _):
        v = cnt_v[pl.ds(i * L, L)]
        cnt_v[pl.ds(i * L, L)] = 1.0 / (v + 1.0)
        return 0
    lax.fori_loop(0, PW // L, invf, 0)

    def chunk_compute(g, b, a):
        def row_body(r, a2):
            inv_b = plsc.load_gather(
                cnt_v, [jnp.full((L,), g * CH + r, jnp.int32)])
            accs = [jnp.zeros((L,), jnp.float32) for _ in range(4)]
            for kd in range(DB):
                f = feat3[b, r, pl.ds(kd * L, L)]
                cc = rows3[b, r, pl.ds(kd * L, L)]
                d = f - cc
                accs[kd % 4] = accs[kd % 4] + d * d
            s4 = (accs[0] + accs[1]) + (accs[2] + accs[3])
            return a2 + s4 * inv_b

        return lax.fori_loop(0, CH, row_body, a)

    def round_body(g, a):
        b = lax.rem(g, NBUF)
        wait(g, b)
        nxt = g + LOOK

        @pl.when(nxt < NCHK)
        def _():
            start(nxt, lax.rem(nxt, NBUF))

        return chunk_compute(g, b, a)

    acc = lax.fori_loop(0, NCHK, round_body, jnp.zeros((L,), jnp.float32))

    acc_buf[...] = acc
    pltpu.sync_copy(acc_buf, out_hbm.at[wid])


@jax.jit
def _sc_center_loss(feat, y2, centers):
    mesh = plsc.VectorSubcoreMesh(core_axis_name="c", subcore_axis_name="s")
    run = pl.kernel(
        _body,
        out_type=jax.ShapeDtypeStruct((NW, L), jnp.float32),
        mesh=mesh,
        scratch_types=[
            pltpu.VMEM_SHARED((HIST,), jnp.float32),
            pltpu.VMEM((NHC, HCHUNK), jnp.int32),
            pltpu.VMEM((PW,), jnp.float32),
            pltpu.VMEM((ZS,), jnp.float32),
            pltpu.VMEM((HCHUNK,), jnp.float32),
            pltpu.VMEM((NBUF, CH, D), jnp.float32),
            pltpu.VMEM((NBUF, CH, D), jnp.float32),
            pltpu.VMEM((L,), jnp.float32),
            pltpu.SemaphoreType.DMA,
            pltpu.SemaphoreType.DMA((NBUF,)),
            pltpu.SemaphoreType.DMA((NBUF,)),
        ],
        compiler_params=pltpu.CompilerParams(needs_layout_passes=False),
    )
    return run(feat, y2, centers)


def kernel(feat, y, centers):
    partials = _sc_center_loss(feat, y.reshape(B // HCHUNK, HCHUNK), centers)
    return 0.5 * jnp.sum(partials)
